# baseline (speedup 1.0000x reference)
import functools

import jax
import jax.numpy as jnp
from jax import lax
from jax.experimental import pallas as pl
from jax.experimental.pallas import tpu as pltpu
from jax.experimental.pallas import tpu_sc as plsc

_B, _S, _D = 256, 2048, 64
_K = 1024
_N = _B * _S
_NW = 32
_TPW = _N // _NW
_CHUNK = 128
_NCH = _TPW // _CHUNK
_CW = _CHUNK * _D


def _sc_body(v_hbm, bounds_hbm, cb_hbm, out_hbm, vbuf, idxT, cbbuf, bbuf, obuf):
    wid = lax.axis_index("s") * 2 + lax.axis_index("c")
    pltpu.sync_copy(cb_hbm, cbbuf)
    pltpu.sync_copy(bounds_hbm, bbuf)
    lane = lax.iota(jnp.int32, 16)
    l128 = lane * 128
    l64 = lane * 64
    base = wid * _TPW

    def chunk_body(g, carry):
        t0 = (base + g * _CHUNK) * _D
        pltpu.sync_copy(v_hbm.at[pl.ds(t0, _CW)], vbuf)

        @plsc.parallel_loop(0, _CHUNK, unroll=2)
        def bin_t(t):
            for db in range(4):
                v = vbuf[pl.ds(t * _D + db * 16, 16)]
                c0 = (v * 1022.0).astype(jnp.int32)
                cc = jnp.clip(c0 + 1, 1, 1022)
                lo = plsc.load_gather(bbuf, [cc - 1])
                hi = plsc.load_gather(bbuf, [cc])
                idx = ((cc - 1) + (lo < v).astype(jnp.int32)
                       + (hi < v).astype(jnp.int32))
                plsc.store_scatter(idxT, [l128 + (db * 2048 + t)], idx)

        def tg_body(tg, c):
            def cc_body(cj, c2):
                pbase = cj * 8

                @plsc.parallel_loop(
                    0, 16, unroll=2,
                    carry=tuple(jnp.zeros((16,), jnp.float32)
                                for _ in range(16)))
                def accs(dq, accs_in):
                    a = list(accs_in)
                    for u in range(4):
                        d = dq * 4 + u
                        iv = idxT[pl.ds(d * 128 + tg * 16, 16)]
                        for j in range(8):
                            w = plsc.load_gather(cbbuf, [iv + (pbase + j) * _K])
                            lo, hi = plsc.unpack(
                                plsc.bitcast(w, jnp.bfloat16),
                                format=plsc.PackFormat.INTERLEAVED)
                            a[2 * j] = a[2 * j] + lo
                            a[2 * j + 1] = a[2 * j + 1] + hi
                    return tuple(a)
                ob = tg * (16 * _D) + 2 * pbase
                for j in range(16):
                    plsc.store_scatter(obuf, [l64 + (ob + j)],
                                       accs[j] * (1.0 / _D))
                return c2
            return lax.fori_loop(0, 4, cc_body, c)
        lax.fori_loop(0, 8, tg_body, 0)

        pltpu.sync_copy(obuf, out_hbm.at[pl.ds(t0, _CW)])
        return carry

    lax.fori_loop(0, _NCH, chunk_body, 0)


_sc_call = functools.partial(
    pl.kernel,
    out_type=jax.ShapeDtypeStruct((_N * _D,), jnp.float32),
    mesh=plsc.VectorSubcoreMesh(core_axis_name="c", subcore_axis_name="s"),
    compiler_params=pltpu.CompilerParams(needs_layout_passes=False),
    scratch_types=[
        pltpu.VMEM((_CW,), jnp.float32),
        pltpu.VMEM((_CW,), jnp.int32),
        pltpu.VMEM((_K * _D // 2,), jnp.int32),
        pltpu.VMEM((_K,), jnp.float32),
        pltpu.VMEM((_CW,), jnp.float32),
    ],
)(_sc_body)


def kernel(x, codebook, bin_bounds):
    B, S, D = x.shape
    mean = jnp.mean(x, axis=1, keepdims=True)
    scaled_x = x / (mean + 1e-06)
    min_val = jnp.min(scaled_x, axis=1, keepdims=True)
    max_val = jnp.max(scaled_x, axis=1, keepdims=True)
    normalized = (scaled_x - min_val) / (max_val - min_val + 1e-06)
    bounds_pad = jnp.concatenate(
        [bin_bounds, jnp.zeros((1,), jnp.float32)])
    bits = jax.lax.bitcast_convert_type(
        codebook.astype(jnp.bfloat16), jnp.uint16).astype(jnp.uint32)
    words = (bits[:, 0::2] | (bits[:, 1::2] << 16)).astype(jnp.int32)
    out = _sc_call(normalized.reshape(-1), bounds_pad, words.T.reshape(-1))
    return out.reshape(B, S, D)


# speedup vs baseline: 1.4338x; 1.4338x over previous
import functools

import jax
import jax.numpy as jnp
from jax import lax
from jax.experimental import pallas as pl
from jax.experimental.pallas import tpu as pltpu
from jax.experimental.pallas import tpu_sc as plsc

_B, _S, _D = 256, 2048, 64
_K = 1024
_N = _B * _S
_NW = 32
_TPW = _N // _NW
_CHUNK = 128
_NCH = _TPW // _CHUNK
_CW = _CHUNK * _D


def _sc_body(v_hbm, bounds_hbm, cb_hbm, out_hbm, vbuf, idxT, cbbuf, bbuf, obuf):
    wid = lax.axis_index("s") * 2 + lax.axis_index("c")
    pltpu.sync_copy(cb_hbm, cbbuf)
    pltpu.sync_copy(bounds_hbm, bbuf)
    lane = lax.iota(jnp.int32, 16)
    l128 = lane * 128
    l64 = lane * 64
    base = wid * _TPW

    def chunk_body(g, carry):
        t0 = (base + g * _CHUNK) * _D
        pltpu.sync_copy(v_hbm.at[pl.ds(t0, _CW)], vbuf)

        @plsc.parallel_loop(0, _CHUNK, unroll=2)
        def bin_t(t):
            for db in range(4):
                v = vbuf[pl.ds(t * _D + db * 16, 16)]
                c0 = (v * 1022.0).astype(jnp.int32)
                cc = jnp.clip(c0 + 1, 1, 1022)
                lo = plsc.load_gather(bbuf, [cc - 1])
                hi = plsc.load_gather(bbuf, [cc])
                idx = ((cc - 1) + (lo < v).astype(jnp.int32)
                       + (hi < v).astype(jnp.int32))
                plsc.store_scatter(idxT, [l128 + (db * 2048 + t)], idx)

        def tg_body(tg, c):
            def cc_body(cj, c2):
                pbase = cj * 8

                def d_body(dq, accs_in):
                    a = list(accs_in)
                    for u in range(4):
                        d = dq * 4 + u
                        iv = idxT[pl.ds(d * 128 + tg * 16, 16)]
                        for j in range(8):
                            w = plsc.load_gather(cbbuf, [iv + (pbase + j) * _K])
                            lo, hi = plsc.unpack(
                                plsc.bitcast(w, jnp.bfloat16),
                                format=plsc.PackFormat.INTERLEAVED)
                            a[2 * j] = a[2 * j] + lo
                            a[2 * j + 1] = a[2 * j + 1] + hi
                    return tuple(a)

                accs = lax.fori_loop(
                    0, 16, d_body,
                    tuple(jnp.zeros((16,), jnp.float32) for _ in range(16)))
                ob = tg * (16 * _D) + 2 * pbase
                for j in range(16):
                    plsc.store_scatter(obuf, [l64 + (ob + j)],
                                       accs[j] * (1.0 / _D))
                return c2
            return lax.fori_loop(0, 4, cc_body, c)
        lax.fori_loop(0, 8, tg_body, 0)

        pltpu.sync_copy(obuf, out_hbm.at[pl.ds(t0, _CW)])
        return carry

    lax.fori_loop(0, _NCH, chunk_body, 0)


_sc_call = functools.partial(
    pl.kernel,
    out_type=jax.ShapeDtypeStruct((_N * _D,), jnp.float32),
    mesh=plsc.VectorSubcoreMesh(core_axis_name="c", subcore_axis_name="s"),
    compiler_params=pltpu.CompilerParams(needs_layout_passes=False),
    scratch_types=[
        pltpu.VMEM((_CW,), jnp.float32),
        pltpu.VMEM((_CW,), jnp.int32),
        pltpu.VMEM((_K * _D // 2,), jnp.int32),
        pltpu.VMEM((_K,), jnp.float32),
        pltpu.VMEM((_CW,), jnp.float32),
    ],
)(_sc_body)


def kernel(x, codebook, bin_bounds):
    B, S, D = x.shape
    mean = jnp.mean(x, axis=1, keepdims=True)
    scaled_x = x / (mean + 1e-06)
    min_val = jnp.min(scaled_x, axis=1, keepdims=True)
    max_val = jnp.max(scaled_x, axis=1, keepdims=True)
    normalized = (scaled_x - min_val) / (max_val - min_val + 1e-06)
    bounds_pad = jnp.concatenate(
        [bin_bounds, jnp.zeros((1,), jnp.float32)])
    bits = jax.lax.bitcast_convert_type(
        codebook.astype(jnp.bfloat16), jnp.uint16).astype(jnp.uint32)
    words = (bits[:, 0::2] | (bits[:, 1::2] << 16)).astype(jnp.int32)
    out = _sc_call(normalized.reshape(-1), bounds_pad, words.T.reshape(-1))
    return out.reshape(B, S, D)


# speedup vs baseline: 1.4443x; 1.0074x over previous
import functools

import jax
import jax.numpy as jnp
from jax import lax
from jax.experimental import pallas as pl
from jax.experimental.pallas import tpu as pltpu
from jax.experimental.pallas import tpu_sc as plsc

_B, _S, _D = 256, 2048, 64
_K = 1024
_N = _B * _S
_NW = 32
_TPW = _N // _NW
_CHUNK = 256
_NCH = _TPW // _CHUNK
_CW = _CHUNK * _D


def _sc_body(v_hbm, bounds_hbm, cb_hbm, out_hbm, vbuf, idxT, cbbuf, bbuf, obuf):
    wid = lax.axis_index("s") * 2 + lax.axis_index("c")
    pltpu.sync_copy(cb_hbm, cbbuf)
    pltpu.sync_copy(bounds_hbm, bbuf)
    lane = lax.iota(jnp.int32, 16)
    lT = lane * _CHUNK
    l64 = lane * 64
    base = wid * _TPW

    def chunk_body(g, carry):
        t0 = (base + g * _CHUNK) * _D
        pltpu.sync_copy(v_hbm.at[pl.ds(t0, _CW)], vbuf)

        @plsc.parallel_loop(0, _CHUNK, unroll=2)
        def bin_t(t):
            for db in range(4):
                v = vbuf[pl.ds(t * _D + db * 16, 16)]
                c0 = (v * 1022.0).astype(jnp.int32)
                cc = jnp.clip(c0 + 1, 1, 1022)
                lo = plsc.load_gather(bbuf, [cc - 1])
                hi = plsc.load_gather(bbuf, [cc])
                idx = ((cc - 1) + (lo < v).astype(jnp.int32)
                       + (hi < v).astype(jnp.int32))
                plsc.store_scatter(
                    idxT, [lT + (db * 16 * _CHUNK + t)], idx)

        def tg_body(tg, c):
            def cc_body(cj, c2):
                pbase = cj * 8

                def d_body(dq, accs_in):
                    a = list(accs_in)
                    for u in range(4):
                        d = dq * 4 + u
                        iv = idxT[pl.ds(d * _CHUNK + tg * 16, 16)]
                        for j in range(8):
                            w = plsc.load_gather(cbbuf, [iv + (pbase + j) * _K])
                            lo, hi = plsc.unpack(
                                plsc.bitcast(w, jnp.bfloat16),
                                format=plsc.PackFormat.INTERLEAVED)
                            a[2 * j] = a[2 * j] + lo
                            a[2 * j + 1] = a[2 * j + 1] + hi
                    return tuple(a)

                accs = lax.fori_loop(
                    0, 16, d_body,
                    tuple(jnp.zeros((16,), jnp.float32) for _ in range(16)))
                ob = tg * (16 * _D) + 2 * pbase
                for j in range(16):
                    plsc.store_scatter(obuf, [l64 + (ob + j)],
                                       accs[j] * (1.0 / _D))
                return c2
            return lax.fori_loop(0, 4, cc_body, c)
        lax.fori_loop(0, _CHUNK // 16, tg_body, 0)

        pltpu.sync_copy(obuf, out_hbm.at[pl.ds(t0, _CW)])
        return carry

    lax.fori_loop(0, _NCH, chunk_body, 0)


_sc_call = functools.partial(
    pl.kernel,
    out_type=jax.ShapeDtypeStruct((_N * _D,), jnp.float32),
    mesh=plsc.VectorSubcoreMesh(core_axis_name="c", subcore_axis_name="s"),
    compiler_params=pltpu.CompilerParams(needs_layout_passes=False),
    scratch_types=[
        pltpu.VMEM((_CW,), jnp.float32),
        pltpu.VMEM((_CW,), jnp.int32),
        pltpu.VMEM((_K * _D // 2,), jnp.int32),
        pltpu.VMEM((_K,), jnp.float32),
        pltpu.VMEM((_CW,), jnp.float32),
    ],
)(_sc_body)


def kernel(x, codebook, bin_bounds):
    B, S, D = x.shape
    mean = jnp.mean(x, axis=1, keepdims=True)
    scaled_x = x / (mean + 1e-06)
    min_val = jnp.min(scaled_x, axis=1, keepdims=True)
    max_val = jnp.max(scaled_x, axis=1, keepdims=True)
    normalized = (scaled_x - min_val) / (max_val - min_val + 1e-06)
    bounds_pad = jnp.concatenate(
        [bin_bounds, jnp.zeros((1,), jnp.float32)])
    bits = jax.lax.bitcast_convert_type(
        codebook.astype(jnp.bfloat16), jnp.uint16).astype(jnp.uint32)
    words = (bits[:, 0::2] | (bits[:, 1::2] << 16)).astype(jnp.int32)
    out = _sc_call(normalized.reshape(-1), bounds_pad, words.T.reshape(-1))
    return out.reshape(B, S, D)


# speedup vs baseline: 1.5522x; 1.0747x over previous
import functools

import jax
import jax.numpy as jnp
from jax import lax
from jax.experimental import pallas as pl
from jax.experimental.pallas import tpu as pltpu
from jax.experimental.pallas import tpu_sc as plsc

_B, _S, _D = 256, 2048, 64
_K = 1024
_N = _B * _S
_NW = 32
_TPW = _N // _NW
_CHUNK = 256
_NCH = _TPW // _CHUNK
_CW = _CHUNK * _D


def _sc_body(v_hbm, bounds_hbm, cb_hbm, out_hbm, vbuf, idxT, cbbuf, bbuf, obuf):
    wid = lax.axis_index("s") * 2 + lax.axis_index("c")
    pltpu.sync_copy(cb_hbm, cbbuf)
    pltpu.sync_copy(bounds_hbm, bbuf)
    lane = lax.iota(jnp.int32, 16)
    lT = lane * _CHUNK
    l64 = lane * 64
    base = wid * _TPW

    def chunk_body(g, carry):
        t0 = (base + g * _CHUNK) * _D
        pltpu.sync_copy(v_hbm.at[pl.ds(t0, _CW)], vbuf)

        @plsc.parallel_loop(0, _CHUNK, unroll=2)
        def bin_t(t):
            for db in range(4):
                v = vbuf[pl.ds(t * _D + db * 16, 16)]
                c0 = (v * 1022.0).astype(jnp.int32)
                cc = jnp.clip(c0 + 1, 1, 1022)
                lo = plsc.load_gather(bbuf, [cc - 1])
                hi = plsc.load_gather(bbuf, [cc])
                idx = ((cc - 1) + (lo < v).astype(jnp.int32)
                       + (hi < v).astype(jnp.int32))
                plsc.store_scatter(
                    idxT, [lT + (db * 16 * _CHUNK + t)], idx)

        def tg_body(tg, c):
            def cc_body(cj, c2):
                pbase = cj * 8

                def d_body(dq, accs_in):
                    a = list(accs_in)
                    ivs = [idxT[pl.ds((dq * 4 + u) * _CHUNK + tg * 16, 16)]
                           for u in range(4)]
                    for j in range(8):
                        col = (pbase + j) * _K
                        w0, w1, w2, w3 = (
                            plsc.bitcast(
                                plsc.load_gather(cbbuf, [ivs[u] + col]),
                                jnp.bfloat16)
                            for u in range(4))
                        b = (w0 + w1) + (w2 + w3)
                        lo, hi = plsc.unpack(
                            b, format=plsc.PackFormat.INTERLEAVED)
                        a[2 * j] = a[2 * j] + lo
                        a[2 * j + 1] = a[2 * j + 1] + hi
                    return tuple(a)

                accs = lax.fori_loop(
                    0, 16, d_body,
                    tuple(jnp.zeros((16,), jnp.float32) for _ in range(16)))
                ob = tg * (16 * _D) + 2 * pbase
                for j in range(16):
                    plsc.store_scatter(obuf, [l64 + (ob + j)],
                                       accs[j] * (1.0 / _D))
                return c2
            return lax.fori_loop(0, 4, cc_body, c)
        lax.fori_loop(0, _CHUNK // 16, tg_body, 0)

        pltpu.sync_copy(obuf, out_hbm.at[pl.ds(t0, _CW)])
        return carry

    lax.fori_loop(0, _NCH, chunk_body, 0)


_sc_call = functools.partial(
    pl.kernel,
    out_type=jax.ShapeDtypeStruct((_N * _D,), jnp.float32),
    mesh=plsc.VectorSubcoreMesh(core_axis_name="c", subcore_axis_name="s"),
    compiler_params=pltpu.CompilerParams(needs_layout_passes=False),
    scratch_types=[
        pltpu.VMEM((_CW,), jnp.float32),
        pltpu.VMEM((_CW,), jnp.int32),
        pltpu.VMEM((_K * _D // 2,), jnp.int32),
        pltpu.VMEM((_K,), jnp.float32),
        pltpu.VMEM((_CW,), jnp.float32),
    ],
)(_sc_body)


def kernel(x, codebook, bin_bounds):
    B, S, D = x.shape
    mean = jnp.mean(x, axis=1, keepdims=True)
    scaled_x = x / (mean + 1e-06)
    min_val = jnp.min(scaled_x, axis=1, keepdims=True)
    max_val = jnp.max(scaled_x, axis=1, keepdims=True)
    normalized = (scaled_x - min_val) / (max_val - min_val + 1e-06)
    bounds_pad = jnp.concatenate(
        [bin_bounds, jnp.zeros((1,), jnp.float32)])
    bits = jax.lax.bitcast_convert_type(
        codebook.astype(jnp.bfloat16), jnp.uint16).astype(jnp.uint32)
    words = (bits[:, 0::2] | (bits[:, 1::2] << 16)).astype(jnp.int32)
    out = _sc_call(normalized.reshape(-1), bounds_pad, words.T.reshape(-1))
    return out.reshape(B, S, D)
